# baseline (speedup 1.0000x reference)
import functools

import jax
import jax.numpy as jnp
from jax.experimental import pallas as pl

B = 2
N1 = 8192
N2 = 8192
KNN = 8
BQ = 256

_BIG = 3e38


def _tc_body(x2_ref, x1_ref, f1_ref, out_ref):
    q = x2_ref[0]
    keys = x1_ref[0] + f1_ref[0]
    f = f1_ref[0]

    k2 = jnp.sum(keys * keys, axis=0, keepdims=True)
    q2 = jnp.sum(q * q, axis=0, keepdims=True)

    qb = q.astype(jnp.bfloat16)
    kb = keys.astype(jnp.bfloat16)
    qk = jax.lax.dot_general(
        qb, kb, (((0,), (0,)), ((), ())),
        preferred_element_type=jnp.float32)
    dist = (q2.T + k2) - 2.0 * qk

    iota = jax.lax.broadcasted_iota(jnp.int32, (1, N1), 1)
    fsum = jnp.zeros((3, BQ), dtype=jnp.float32)
    for _ in range(KNN):
        m = jnp.min(dist, axis=1, keepdims=True)
        am = jnp.min(jnp.where(dist == m, iota, N1), axis=1,
                     keepdims=True)
        onehot = (iota == am)
        fsum = fsum + jax.lax.dot_general(
            f, onehot.astype(jnp.float32), (((1,), (1,)), ((), ())),
            preferred_element_type=jnp.float32)
        dist = jnp.where(onehot, _BIG, dist)

    out_ref[0] = q - fsum * jnp.float32(1.0 / KNN)


def kernel(xyz1, xyz2, flow1, K):
    del K
    grid = (B, N2 // BQ)
    out = pl.pallas_call(
        _tc_body,
        grid=grid,
        in_specs=[
            pl.BlockSpec((1, 3, BQ), lambda b, i: (b, 0, i)),
            pl.BlockSpec((1, 3, N1), lambda b, i: (b, 0, 0)),
            pl.BlockSpec((1, 3, N1), lambda b, i: (b, 0, 0)),
        ],
        out_specs=pl.BlockSpec((1, 3, BQ), lambda b, i: (b, 0, i)),
        out_shape=jax.ShapeDtypeStruct((B, 3, N2), jnp.float32),
    )(xyz2, xyz1, flow1)
    return out


# speedup vs baseline: 1.0404x; 1.0404x over previous
import functools

import jax
import jax.numpy as jnp
from jax.experimental import pallas as pl

B = 2
N1 = 8192
N2 = 8192
KNN = 8
BQ = 256

_BIG = 3e38


def _tc_body(x2_ref, x1_ref, f1_ref, out_ref):
    q = x2_ref[0]
    keys = x1_ref[0] + f1_ref[0]
    f = f1_ref[0]

    k2 = jnp.sum(keys * keys, axis=0, keepdims=True)
    q2 = jnp.sum(q * q, axis=0, keepdims=True)

    qb = q.astype(jnp.bfloat16)
    kb = keys.astype(jnp.bfloat16)
    qk = jax.lax.dot_general(
        qb, kb, (((0,), (0,)), ((), ())),
        preferred_element_type=jnp.float32)
    dist = (q2.T + k2) - 2.0 * qk

    iota = jax.lax.broadcasted_iota(jnp.int32, (1, N1), 1)
    sel = jnp.zeros((BQ, N1), dtype=jnp.float32)
    for _ in range(KNN):
        m = jnp.min(dist, axis=1, keepdims=True)
        am = jnp.min(jnp.where(dist == m, iota, N1), axis=1,
                     keepdims=True)
        onehot = (iota == am)
        sel = sel + onehot.astype(jnp.float32)
        dist = jnp.where(onehot, _BIG, dist)

    fsum = jax.lax.dot_general(
        f, sel, (((1,), (1,)), ((), ())),
        preferred_element_type=jnp.float32)
    out_ref[0] = q - fsum * jnp.float32(1.0 / KNN)


def kernel(xyz1, xyz2, flow1, K):
    del K
    grid = (B, N2 // BQ)
    out = pl.pallas_call(
        _tc_body,
        grid=grid,
        in_specs=[
            pl.BlockSpec((1, 3, BQ), lambda b, i: (b, 0, i)),
            pl.BlockSpec((1, 3, N1), lambda b, i: (b, 0, 0)),
            pl.BlockSpec((1, 3, N1), lambda b, i: (b, 0, 0)),
        ],
        out_specs=pl.BlockSpec((1, 3, BQ), lambda b, i: (b, 0, i)),
        out_shape=jax.ShapeDtypeStruct((B, 3, N2), jnp.float32),
    )(xyz2, xyz1, flow1)
    return out


# speedup vs baseline: 3.4430x; 3.3094x over previous
import jax
import jax.numpy as jnp
from jax.experimental import pallas as pl

B = 2
N1 = 8192
N2 = 8192
KNN = 8
BQ = 256
NCH = 64

_BIG = 3e38


def _top2_tree(v):
    nch = v.shape[1]
    m1, m2 = v, jnp.full_like(v, _BIG)
    while m1.shape[1] > 1:
        h = m1.shape[1] // 2
        a1, b1 = m1[:, :h], m1[:, h:]
        a2, b2 = m2[:, :h], m2[:, h:]
        m1 = jnp.minimum(a1, b1)
        m2 = jnp.minimum(jnp.maximum(a1, b1), jnp.minimum(a2, b2))
    return m1[:, 0], m2[:, 0]


def _tc_body(x2_ref, x1_ref, f1_ref, out_ref):
    q = x2_ref[0]
    keys = x1_ref[0] + f1_ref[0]
    f = f1_ref[0]

    k2 = jnp.sum(keys * keys, axis=0, keepdims=True)
    q2 = jnp.sum(q * q, axis=0, keepdims=True)

    qb = q.astype(jnp.bfloat16)
    kb = keys.astype(jnp.bfloat16)
    qk = jax.lax.dot_general(
        qb, kb, (((0,), (0,)), ((), ())),
        preferred_element_type=jnp.float32)
    dist = (q2.T + k2) - 2.0 * qk

    m1, m2 = _top2_tree(dist.reshape(BQ, NCH, 128))
    cand = jnp.concatenate([m1, m2], axis=1)
    t = jnp.float32(0)
    for _ in range(KNN):
        t = jnp.min(cand, axis=1, keepdims=True)
        cand = jnp.where(cand == t, _BIG, cand)

    sel = (dist <= t).astype(jnp.float32)
    cnt = jnp.sum(sel, axis=1, keepdims=True)
    fsum = jax.lax.dot_general(
        f, sel, (((1,), (1,)), ((), ())),
        preferred_element_type=jnp.float32)
    out_ref[0] = q - fsum * (1.0 / cnt).T


def kernel(xyz1, xyz2, flow1, K):
    del K
    grid = (B, N2 // BQ)
    out = pl.pallas_call(
        _tc_body,
        grid=grid,
        in_specs=[
            pl.BlockSpec((1, 3, BQ), lambda b, i: (b, 0, i)),
            pl.BlockSpec((1, 3, N1), lambda b, i: (b, 0, 0)),
            pl.BlockSpec((1, 3, N1), lambda b, i: (b, 0, 0)),
        ],
        out_specs=pl.BlockSpec((1, 3, BQ), lambda b, i: (b, 0, i)),
        out_shape=jax.ShapeDtypeStruct((B, 3, N2), jnp.float32),
    )(xyz2, xyz1, flow1)
    return out


# speedup vs baseline: 3.7587x; 1.0917x over previous
import jax
import jax.numpy as jnp
from jax.experimental import pallas as pl

B = 2
N1 = 8192
N2 = 8192
KNN = 8
BQ = 256
NCH = 64

_BIG = 3e38


def _top2_tree(v):
    nch = v.shape[1]
    m1, m2 = v, jnp.full_like(v, _BIG)
    while m1.shape[1] > 1:
        h = m1.shape[1] // 2
        a1, b1 = m1[:, :h], m1[:, h:]
        a2, b2 = m2[:, :h], m2[:, h:]
        m1 = jnp.minimum(a1, b1)
        m2 = jnp.minimum(jnp.maximum(a1, b1), jnp.minimum(a2, b2))
    return m1[:, 0], m2[:, 0]


def _tc_body(x2_ref, x1_ref, f1_ref, out_ref):
    q = x2_ref[0]
    keys = x1_ref[0] + f1_ref[0]
    f = f1_ref[0]

    k2h = 0.5 * jnp.sum(keys * keys, axis=0, keepdims=True)

    qb = q.astype(jnp.bfloat16)
    kb = keys.astype(jnp.bfloat16)
    qk = jax.lax.dot_general(
        qb, kb, (((0,), (0,)), ((), ())),
        preferred_element_type=jnp.float32)
    dist = k2h - qk

    m1, m2 = _top2_tree(dist.reshape(BQ, NCH, 128))
    cand = jnp.concatenate([m1, m2], axis=1)
    cand0 = cand
    t = jnp.float32(0)
    for _ in range(KNN):
        t = jnp.min(cand, axis=1, keepdims=True)
        cand = jnp.where(cand == t, _BIG, cand)

    sel = (dist <= t).astype(jnp.float32)
    cnt = jnp.sum((cand0 <= t).astype(jnp.float32), axis=1,
                  keepdims=True)
    fsum = jax.lax.dot_general(
        f, sel, (((1,), (1,)), ((), ())),
        preferred_element_type=jnp.float32)
    out_ref[0] = q - fsum * (1.0 / cnt).T


def kernel(xyz1, xyz2, flow1, K):
    del K
    grid = (B, N2 // BQ)
    out = pl.pallas_call(
        _tc_body,
        grid=grid,
        in_specs=[
            pl.BlockSpec((1, 3, BQ), lambda b, i: (b, 0, i)),
            pl.BlockSpec((1, 3, N1), lambda b, i: (b, 0, 0)),
            pl.BlockSpec((1, 3, N1), lambda b, i: (b, 0, 0)),
        ],
        out_specs=pl.BlockSpec((1, 3, BQ), lambda b, i: (b, 0, i)),
        out_shape=jax.ShapeDtypeStruct((B, 3, N2), jnp.float32),
    )(xyz2, xyz1, flow1)
    return out


# speedup vs baseline: 4.8851x; 1.2997x over previous
import jax
import jax.numpy as jnp
from jax.experimental import pallas as pl

B = 2
N1 = 8192
N2 = 8192
KNN = 8
BQ = 512
NCH = 64

_BIG = 3e38


def _top2_tree(v):
    nch = v.shape[1]
    m1, m2 = v, jnp.full_like(v, _BIG)
    while m1.shape[1] > 1:
        h = m1.shape[1] // 2
        a1, b1 = m1[:, :h], m1[:, h:]
        a2, b2 = m2[:, :h], m2[:, h:]
        m1 = jnp.minimum(a1, b1)
        m2 = jnp.minimum(jnp.maximum(a1, b1), jnp.minimum(a2, b2))
    return m1[:, 0], m2[:, 0]


def _tc_body(x2_ref, x1_ref, f1_ref, out_ref):
    q = x2_ref[0]
    keys = x1_ref[0] + f1_ref[0]
    f = f1_ref[0]

    k2h = 0.5 * jnp.sum(keys * keys, axis=0, keepdims=True)

    qb = q.astype(jnp.bfloat16)
    kb = keys.astype(jnp.bfloat16)
    qk = jax.lax.dot_general(
        qb, kb, (((0,), (0,)), ((), ())),
        preferred_element_type=jnp.float32)
    dist = k2h - qk

    m1, m2 = _top2_tree(dist.reshape(BQ, NCH, 128))
    cand = jnp.concatenate([m1, m2], axis=1)
    cand0 = cand
    t = jnp.float32(0)
    for _ in range(KNN):
        t = jnp.min(cand, axis=1, keepdims=True)
        cand = jnp.where(cand == t, _BIG, cand)

    sel = (dist <= t).astype(jnp.float32)
    cnt = jnp.sum((cand0 <= t).astype(jnp.float32), axis=1,
                  keepdims=True)
    fsum = jax.lax.dot_general(
        f, sel, (((1,), (1,)), ((), ())),
        preferred_element_type=jnp.float32)
    out_ref[0] = q - fsum * (1.0 / cnt).T


def kernel(xyz1, xyz2, flow1, K):
    del K
    grid = (B, N2 // BQ)
    out = pl.pallas_call(
        _tc_body,
        grid=grid,
        in_specs=[
            pl.BlockSpec((1, 3, BQ), lambda b, i: (b, 0, i)),
            pl.BlockSpec((1, 3, N1), lambda b, i: (b, 0, 0)),
            pl.BlockSpec((1, 3, N1), lambda b, i: (b, 0, 0)),
        ],
        out_specs=pl.BlockSpec((1, 3, BQ), lambda b, i: (b, 0, i)),
        out_shape=jax.ShapeDtypeStruct((B, 3, N2), jnp.float32),
    )(xyz2, xyz1, flow1)
    return out


# speedup vs baseline: 5.2925x; 1.0834x over previous
import jax
import jax.numpy as jnp
from jax.experimental import pallas as pl

B = 2
N1 = 8192
N2 = 8192
KNN = 8
BQ = 1024
NCH = 64

_BIG = 3e38


def _top2_tree(v):
    nch = v.shape[1]
    m1, m2 = v, jnp.full_like(v, _BIG)
    while m1.shape[1] > 1:
        h = m1.shape[1] // 2
        a1, b1 = m1[:, :h], m1[:, h:]
        a2, b2 = m2[:, :h], m2[:, h:]
        m1 = jnp.minimum(a1, b1)
        m2 = jnp.minimum(jnp.maximum(a1, b1), jnp.minimum(a2, b2))
    return m1[:, 0], m2[:, 0]


def _tc_body(x2_ref, x1_ref, f1_ref, out_ref):
    q = x2_ref[0]
    keys = x1_ref[0] + f1_ref[0]
    f = f1_ref[0]

    k2h = 0.5 * jnp.sum(keys * keys, axis=0, keepdims=True)

    qb = q.astype(jnp.bfloat16)
    kb = keys.astype(jnp.bfloat16)
    qk = jax.lax.dot_general(
        qb, kb, (((0,), (0,)), ((), ())),
        preferred_element_type=jnp.float32)
    dist = k2h - qk

    m1, m2 = _top2_tree(dist.reshape(BQ, NCH, 128))
    cand = jnp.concatenate([m1, m2], axis=1)
    cand0 = cand
    t = jnp.float32(0)
    for _ in range(KNN):
        t = jnp.min(cand, axis=1, keepdims=True)
        cand = jnp.where(cand == t, _BIG, cand)

    sel = (dist <= t).astype(jnp.float32)
    cnt = jnp.sum((cand0 <= t).astype(jnp.float32), axis=1,
                  keepdims=True)
    fsum = jax.lax.dot_general(
        f, sel, (((1,), (1,)), ((), ())),
        preferred_element_type=jnp.float32)
    out_ref[0] = q - fsum * (1.0 / cnt).T


def kernel(xyz1, xyz2, flow1, K):
    del K
    grid = (B, N2 // BQ)
    out = pl.pallas_call(
        _tc_body,
        grid=grid,
        in_specs=[
            pl.BlockSpec((1, 3, BQ), lambda b, i: (b, 0, i)),
            pl.BlockSpec((1, 3, N1), lambda b, i: (b, 0, 0)),
            pl.BlockSpec((1, 3, N1), lambda b, i: (b, 0, 0)),
        ],
        out_specs=pl.BlockSpec((1, 3, BQ), lambda b, i: (b, 0, i)),
        out_shape=jax.ShapeDtypeStruct((B, 3, N2), jnp.float32),
    )(xyz2, xyz1, flow1)
    return out


# speedup vs baseline: 6.5636x; 1.2402x over previous
import jax
import jax.numpy as jnp
from jax.experimental import pallas as pl

B = 2
N1 = 8192
N2 = 8192
KNN = 8
BQ = 1024
NCH = 64

_BIG = 3e38


def _min_tree(v, stop_h):
    while v.shape[1] > stop_h:
        h = v.shape[1] // 2
        v = jnp.minimum(v[:, :h], v[:, h:])
    return v


def _tc_body(x2_ref, x1_ref, f1_ref, out_ref):
    q = x2_ref[0]
    keys = x1_ref[0] + f1_ref[0]
    f = f1_ref[0]

    k2h = 0.5 * jnp.sum(keys * keys, axis=0, keepdims=True)

    qb = q.astype(jnp.bfloat16)
    kb = keys.astype(jnp.bfloat16)
    qk = jax.lax.dot_general(
        qb, kb, (((0,), (0,)), ((), ())),
        preferred_element_type=jnp.float32)
    dist = k2h - qk

    m8 = _min_tree(dist.reshape(BQ, NCH, 128), 8)
    m4 = jnp.minimum(m8[:, :4], m8[:, 4:])
    cand = m4.reshape(BQ, 512)
    cand0 = cand
    t = jnp.float32(0)
    for _ in range(KNN):
        t = jnp.min(cand, axis=1, keepdims=True)
        cand = jnp.where(cand == t, _BIG, cand)

    sel = (dist <= t).astype(jnp.float32)
    cnt = jnp.sum((cand0 <= t).astype(jnp.float32), axis=1,
                  keepdims=True)
    fsum = jax.lax.dot_general(
        f, sel, (((1,), (1,)), ((), ())),
        preferred_element_type=jnp.float32)
    out_ref[0] = q - fsum * (1.0 / cnt).T


def kernel(xyz1, xyz2, flow1, K):
    del K
    grid = (B, N2 // BQ)
    out = pl.pallas_call(
        _tc_body,
        grid=grid,
        in_specs=[
            pl.BlockSpec((1, 3, BQ), lambda b, i: (b, 0, i)),
            pl.BlockSpec((1, 3, N1), lambda b, i: (b, 0, 0)),
            pl.BlockSpec((1, 3, N1), lambda b, i: (b, 0, 0)),
        ],
        out_specs=pl.BlockSpec((1, 3, BQ), lambda b, i: (b, 0, i)),
        out_shape=jax.ShapeDtypeStruct((B, 3, N2), jnp.float32),
    )(xyz2, xyz1, flow1)
    return out


# speedup vs baseline: 6.8386x; 1.0419x over previous
import jax
import jax.numpy as jnp
from jax.experimental import pallas as pl

B = 2
N1 = 8192
N2 = 8192
KNN = 8
BQ = 1024
NCH = 64

_BIG = 3e38


def _min_tree(v, stop_h):
    while v.shape[1] > stop_h:
        h = v.shape[1] // 2
        v = jnp.minimum(v[:, :h], v[:, h:])
    return v


def _tc_body(x2_ref, x1_ref, f1_ref, out_ref):
    q = x2_ref[0]
    keys = x1_ref[0] + f1_ref[0]
    f = f1_ref[0]

    k2h = 0.5 * jnp.sum(keys * keys, axis=0, keepdims=True)

    s1 = k2h.astype(jnp.bfloat16)
    r1 = k2h - s1.astype(jnp.float32)
    s2 = r1.astype(jnp.bfloat16)
    s3 = (r1 - s2.astype(jnp.float32)).astype(jnp.bfloat16)
    qb = (-q).astype(jnp.bfloat16)
    kb = keys.astype(jnp.bfloat16)
    lhs = jnp.concatenate(
        [qb, jnp.ones((3, BQ), dtype=jnp.bfloat16)], axis=0)
    rhs = jnp.concatenate([kb, s1, s2, s3], axis=0)
    dist = jax.lax.dot_general(
        lhs, rhs, (((0,), (0,)), ((), ())),
        preferred_element_type=jnp.float32)

    m8 = _min_tree(dist.reshape(BQ, NCH, 128), 8)
    m4 = jnp.minimum(m8[:, :4], m8[:, 4:])
    cand = m4.reshape(BQ, 512)
    cand0 = cand
    t = jnp.float32(0)
    for _ in range(KNN):
        t = jnp.min(cand, axis=1, keepdims=True)
        cand = jnp.where(cand == t, _BIG, cand)

    sel = (dist <= t).astype(jnp.bfloat16)
    cnt = jnp.sum((cand0 <= t).astype(jnp.float32), axis=1,
                  keepdims=True)
    fsum = jax.lax.dot_general(
        f.astype(jnp.bfloat16), sel, (((1,), (1,)), ((), ())),
        preferred_element_type=jnp.float32)
    out_ref[0] = q - fsum * (1.0 / cnt).T


def kernel(xyz1, xyz2, flow1, K):
    del K
    grid = (B, N2 // BQ)
    out = pl.pallas_call(
        _tc_body,
        grid=grid,
        in_specs=[
            pl.BlockSpec((1, 3, BQ), lambda b, i: (b, 0, i)),
            pl.BlockSpec((1, 3, N1), lambda b, i: (b, 0, 0)),
            pl.BlockSpec((1, 3, N1), lambda b, i: (b, 0, 0)),
        ],
        out_specs=pl.BlockSpec((1, 3, BQ), lambda b, i: (b, 0, i)),
        out_shape=jax.ShapeDtypeStruct((B, 3, N2), jnp.float32),
    )(xyz2, xyz1, flow1)
    return out
